# baseline (speedup 1.0000x reference)
import functools

import jax
import jax.numpy as jnp
from jax import lax
from jax.experimental import pallas as pl
from jax.experimental.pallas import tpu as pltpu
from jax.experimental.pallas import tpu_sc as plsc

NC = 2
NS = 16
NW = NC * NS
SUB = 80


def _weights_body(c0, c1, c2, b0, b1, b2, o0, o1, o2):
    for c_ref, b_ref, o_ref in ((c0, b0, o0), (c1, b1, o1), (c2, b2, o2)):
        R, B = c_ref.shape
        for r in range(R):
            acc = b_ref[0] * c_ref[r, 0]
            for b in range(1, B):
                acc = acc + b_ref[b] * c_ref[r, b]
            o_ref[r] = acc


def _make_weights(comp0, bases0, comp1, bases1, comp2, bases2):
    outs = [
        jax.ShapeDtypeStruct((c.shape[0], b.shape[1], b.shape[2]),
                             jnp.float32)
        for c, b in ((comp0, bases0), (comp1, bases1), (comp2, bases2))
    ]
    smem = pl.BlockSpec(memory_space=pltpu.SMEM)
    vmem = pl.BlockSpec()
    return pl.pallas_call(
        _weights_body,
        out_shape=outs,
        in_specs=[smem, smem, smem, vmem, vmem, vmem],
    )(comp0, comp1, comp2, bases0, bases1, bases2)


def _mm0_body(x_ref, wt_ref, out_ref):
    r = jnp.dot(x_ref[...], wt_ref[0], preferred_element_type=jnp.float32)
    out_ref[...] = r.astype(jnp.bfloat16)


def _mm0(x, wt, block_n):
    n, d = x.shape
    nr, _, ho = wt.shape
    nblk = n // block_n
    return pl.pallas_call(
        _mm0_body,
        grid=(nblk, nr),
        in_specs=[
            pl.BlockSpec((block_n, d), lambda i, r: (i, 0)),
            pl.BlockSpec((1, d, ho), lambda i, r: (r, 0, 0)),
        ],
        out_specs=pl.BlockSpec((block_n, ho), lambda i, r: (r * nblk + i, 0)),
        out_shape=jax.ShapeDtypeStruct((n * nr, ho), jnp.bfloat16),
    )(x, wt)


def _mm12_body(p_ref, bias_ref, wt_ref, out_ref):
    act = jnp.maximum(p_ref[0].astype(jnp.float32)
                      + p_ref[1].astype(jnp.float32) + bias_ref[...], 0.0)
    r = jnp.dot(act, wt_ref[0], preferred_element_type=jnp.float32)
    out_ref[...] = r.astype(jnp.bfloat16)


def _mm12(part, bias, wt, block_n):
    _, n, h = part.shape
    nr, _, ho = wt.shape
    nblk = n // block_n
    return pl.pallas_call(
        _mm12_body,
        grid=(nblk, nr),
        in_specs=[
            pl.BlockSpec((2, block_n, h), lambda i, r: (0, i, 0)),
            pl.BlockSpec((1, h), lambda i, r: (0, 0)),
            pl.BlockSpec((1, h, ho), lambda i, r: (r, 0, 0)),
        ],
        out_specs=pl.BlockSpec((block_n, ho), lambda i, r: (r * nblk + i, 0)),
        out_shape=jax.ShapeDtypeStruct((n * nr, ho), jnp.bfloat16),
    )(part, bias.reshape(1, h), wt)


def _make_prep(E, R, N):
    EP = E // NW
    NSUB = EP // SUB
    EPA = (EP // 128 + 1) * 128
    mesh = plsc.VectorSubcoreMesh(core_axis_name="c", subcore_axis_name="s")

    @functools.partial(
        pl.kernel,
        mesh=mesh,
        out_type=(
            jax.ShapeDtypeStruct((NW, NSUB, SUB), jnp.int32),
            jax.ShapeDtypeStruct((NW, NSUB, SUB), jnp.int32),
        ),
        scratch_types=[
            pltpu.VMEM((EPA,), jnp.int32),
            pltpu.VMEM((EPA,), jnp.int32),
            pltpu.VMEM((EP,), jnp.int32),
            pltpu.VMEM((NSUB, SUB), jnp.int32),
            pltpu.VMEM((NSUB, SUB), jnp.int32),
        ],
    )
    def prep(ei_hbm, ety_hbm, gidx_out, dst_out,
             srcv, dstv, etyv, gidx2d, dst2d):
        wid = lax.axis_index("c") * NS + lax.axis_index("s")
        base = wid * EP
        rem = lax.rem(base, 128)
        al = pl.multiple_of(base - rem, 128)
        pltpu.sync_copy(ei_hbm.at[0].at[pl.ds(al, EPA)], srcv)
        pltpu.sync_copy(ei_hbm.at[1].at[pl.ds(al, EPA)], dstv)
        pltpu.sync_copy(ety_hbm.at[pl.ds(base, EP)], etyv)

        @pl.loop(0, NSUB)
        def _(j):
            for k in range(SUB // 16):
                off = j * SUB + k * 16
                s = srcv[pl.ds(off + rem, 16)]
                t = etyv[pl.ds(off, 16)]
                gidx2d[j, pl.ds(k * 16, 16)] = t * N + s
                dst2d[j, pl.ds(k * 16, 16)] = dstv[pl.ds(off + rem, 16)]

        pltpu.sync_copy(gidx2d, gidx_out.at[wid])
        pltpu.sync_copy(dst2d, dst_out.at[wid])

    return prep


def _make_scatter(H, N, E):
    EP = E // NW
    NSUB = EP // SUB
    NT8 = ((N // NS) + 7) // 8 * 8
    NTL = N - NT8 * (NS - 1)
    assert NTL > 0 and NTL % 8 == 0
    mesh = plsc.VectorSubcoreMesh(core_axis_name="c", subcore_axis_name="s")

    @functools.partial(
        pl.kernel,
        mesh=mesh,
        out_type=jax.ShapeDtypeStruct((NC, N, H), jnp.bfloat16),
        scratch_types=[
            pltpu.VMEM((NSUB, SUB), jnp.int32),
            pltpu.VMEM((NSUB, SUB), jnp.int32),
            pltpu.VMEM((16, SUB, H), jnp.bfloat16),
            pltpu.VMEM((320, H), jnp.bfloat16),
            pltpu.VMEM_SHARED((N, H), jnp.bfloat16),
            pltpu.SemaphoreType.DMA,
            pltpu.SemaphoreType.DMA,
        ],
        compiler_params=pltpu.CompilerParams(use_tc_tiling_on_sc=False),
    )
    def scatter(table_hbm, gidx_hbm, dst_hbm, out_hbm,
                gidx2d, dst2d, rows, zbuf, acc, sem, psem):
        cid = lax.axis_index("c")
        sid = lax.axis_index("s")
        wid = cid * NS + sid

        pltpu.sync_copy(gidx_hbm.at[wid], gidx2d)
        pltpu.sync_copy(dst_hbm.at[wid], dst2d)

        ZB = zbuf.shape[0]
        assert ZB <= NT8 <= 2 * ZB and NTL <= 2 * ZB
        assert (NT8 - ZB) % 8 == 0 and (NTL - ZB) % 8 == 0

        @pl.loop(0, ZB)
        def _(i):
            for k in range(H // 32):
                zbuf[i, pl.ds(k * 32, 32)] = jnp.zeros((32,), jnp.bfloat16)

        @pl.when(sid < NS - 1)
        def _():
            pltpu.sync_copy(zbuf, acc.at[pl.ds(sid * NT8, ZB)])
            pltpu.sync_copy(zbuf.at[pl.ds(0, NT8 - ZB)],
                            acc.at[pl.ds(sid * NT8 + ZB, NT8 - ZB)])

        @pl.when(sid == NS - 1)
        def _():
            pltpu.sync_copy(zbuf, acc.at[pl.ds((NS - 1) * NT8, ZB)])
            pltpu.sync_copy(zbuf.at[pl.ds(0, NTL - ZB)],
                            acc.at[pl.ds((NS - 1) * NT8 + ZB, NTL - ZB)])
        plsc.subcore_barrier()

        def fire_g(j, b):
            pltpu.async_copy(table_hbm.at[gidx2d.at[j]], rows.at[b], sem)

        def drain_g(j, b):
            pltpu.make_async_copy(table_hbm.at[gidx2d.at[j]], rows.at[b],
                                  sem).wait()

        def fire_p(j, b):
            pltpu.async_copy(rows.at[b], acc.at[dst2d.at[j]], psem, add=True)

        def drain_p(j, b):
            pltpu.make_async_copy(rows.at[b], acc.at[dst2d.at[j]],
                                  psem).wait()

        NB = rows.shape[0]
        TAIL = NSUB % NB
        assert NSUB >= 2 * NB
        for b in range(NB):
            fire_g(b, b)

        @pl.loop(0, NSUB // NB - 1)
        def _(jj):
            j0 = jj * NB
            for b in range(NB):
                drain_g(j0 + b, b)
                fire_p(j0 + b, b)
            for b in range(NB):
                drain_p(j0 + b, b)
                fire_g(j0 + NB + b, b)

        j0 = NSUB - TAIL - NB
        for b in range(NB):
            drain_g(j0 + b, b)
            fire_p(j0 + b, b)
        for b in range(NB):
            drain_p(j0 + b, b)
            if b < TAIL:
                fire_g(j0 + NB + b, b)
        for b in range(TAIL):
            drain_g(j0 + NB + b, b)
            fire_p(j0 + NB + b, b)
        for b in range(TAIL):
            drain_p(j0 + NB + b, b)
        plsc.subcore_barrier()

        @pl.when(sid < NS - 1)
        def _():
            pltpu.sync_copy(acc.at[pl.ds(sid * NT8, NT8)],
                            out_hbm.at[cid].at[pl.ds(sid * NT8, NT8)])

        @pl.when(sid == NS - 1)
        def _():
            pltpu.sync_copy(acc.at[pl.ds((NS - 1) * NT8, NTL)],
                            out_hbm.at[cid].at[pl.ds((NS - 1) * NT8, NTL)])

    return scatter


def _head_body(p_ref, bias_ref, gw_ref, gb_ref, f1w_ref, f1b_ref,
               f2w_ref, f2b_ref, f3w_ref, f3b_ref, out_ref):
    h = (p_ref[0].astype(jnp.float32) + p_ref[1].astype(jnp.float32)
         + bias_ref[...])
    g = jnp.sum(h * gw_ref[...], axis=1, keepdims=True) + gb_ref[0, 0]
    m = jnp.max(g)
    e = jnp.exp(g - m)
    w = e / jnp.sum(e)
    ro = jnp.sum(h * w, axis=0, keepdims=True)
    z = jnp.maximum(jnp.dot(ro, f1w_ref[...],
                            preferred_element_type=jnp.float32)
                    + f1b_ref[...], 0.0)
    z = jnp.maximum(jnp.dot(z, f2w_ref[...],
                            preferred_element_type=jnp.float32)
                    + f2b_ref[...], 0.0)
    z = jnp.dot(z, f3w_ref[...], preferred_element_type=jnp.float32) \
        + f3b_ref[...]
    out_ref[...] = 1.0 / (1.0 + jnp.exp(-z))


def _head(part, bias, gate_w, gate_b, fc1_w, fc1_b, fc2_w, fc2_b, fc3_w, fc3_b):
    _, n, h = part.shape
    return pl.pallas_call(
        _head_body,
        out_shape=jax.ShapeDtypeStruct((1, 1), jnp.float32),
    )(part, bias.reshape(1, h), gate_w.reshape(1, h), gate_b.reshape(1, 1),
      fc1_w, fc1_b.reshape(1, -1), fc2_w, fc2_b.reshape(1, -1),
      fc3_w, fc3_b.reshape(1, 1))


def kernel(x, edge_index, etype, bases0, comp0, bias0, bases1, comp1, bias1,
           bases2, comp2, bias2, gate_w, gate_b, fc1_w, fc1_b, fc2_w, fc2_b,
           fc3_w, fc3_b):
    n, d = x.shape
    e = etype.shape[0]
    r = comp0.shape[0]
    h = bases0.shape[2]
    o = bases2.shape[2]

    wt0, wt1, wt2 = _make_weights(comp0, bases0, comp1, bases1, comp2, bases2)
    gidx2d, dst2d = _make_prep(e, r, n)(edge_index.astype(jnp.int32),
                                        etype.astype(jnp.int32))
    scatter = _make_scatter(h, n, e)

    t0 = _mm0(x, wt0, block_n=2000)
    p0 = scatter(t0, gidx2d, dst2d)
    t1 = _mm12(p0, bias0, wt1, block_n=2000)
    p1 = scatter(t1, gidx2d, dst2d)
    t2 = _mm12(p1, bias1, wt2, block_n=2000)
    p2 = scatter(t2, gidx2d, dst2d)

    out = _head(p2, bias2, gate_w, gate_b, fc1_w, fc1_b, fc2_w, fc2_b,
                fc3_w, fc3_b)
    return out.reshape((1,))


# speedup vs baseline: 1.3099x; 1.3099x over previous
import functools

import jax
import jax.numpy as jnp
from jax import lax
from jax.experimental import pallas as pl
from jax.experimental.pallas import tpu as pltpu
from jax.experimental.pallas import tpu_sc as plsc

NC = 2
NS = 16
NW = NC * NS
SUB = 80


def _weights_body(c0, c1, c2, b0, b1, b2, o0, o1, o2):
    for c_ref, b_ref, o_ref in ((c0, b0, o0), (c1, b1, o1), (c2, b2, o2)):
        R, B = c_ref.shape
        H = b_ref.shape[2]
        for r in range(R):
            acc = b_ref[0] * c_ref[r, 0]
            for b in range(1, B):
                acc = acc + b_ref[b] * c_ref[r, b]
            o_ref[:, r * H:(r + 1) * H] = acc


def _make_weights(comp0, bases0, comp1, bases1, comp2, bases2):
    outs = [
        jax.ShapeDtypeStruct((b.shape[1], c.shape[0] * b.shape[2]),
                             jnp.float32)
        for c, b in ((comp0, bases0), (comp1, bases1), (comp2, bases2))
    ]
    smem = pl.BlockSpec(memory_space=pltpu.SMEM)
    vmem = pl.BlockSpec()
    return pl.pallas_call(
        _weights_body,
        out_shape=outs,
        in_specs=[smem, smem, smem, vmem, vmem, vmem],
    )(comp0, comp1, comp2, bases0, bases1, bases2)


def _mm0_body(x_ref, wt_ref, out_ref):
    r = jnp.dot(x_ref[...], wt_ref[...], preferred_element_type=jnp.float32)
    out_ref[...] = r.astype(jnp.bfloat16)


def _mm0(x, wt, block_n):
    n, d = x.shape
    ro = wt.shape[1]
    grid = (n // block_n,)
    return pl.pallas_call(
        _mm0_body,
        grid=grid,
        in_specs=[
            pl.BlockSpec((block_n, d), lambda i: (i, 0)),
            pl.BlockSpec((d, ro), lambda i: (0, 0)),
        ],
        out_specs=pl.BlockSpec((block_n, ro), lambda i: (i, 0)),
        out_shape=jax.ShapeDtypeStruct((n, ro), jnp.bfloat16),
    )(x, wt)


def _mm12_body(p_ref, bias_ref, wt_ref, out_ref):
    act = jnp.maximum(p_ref[0].astype(jnp.float32)
                      + p_ref[1].astype(jnp.float32) + bias_ref[...], 0.0)
    r = jnp.dot(act, wt_ref[...], preferred_element_type=jnp.float32)
    out_ref[...] = r.astype(jnp.bfloat16)


def _mm12(part, bias, wt, block_n):
    _, n, h = part.shape
    ro = wt.shape[1]
    grid = (n // block_n,)
    return pl.pallas_call(
        _mm12_body,
        grid=grid,
        in_specs=[
            pl.BlockSpec((2, block_n, h), lambda i: (0, i, 0)),
            pl.BlockSpec((1, h), lambda i: (0, 0)),
            pl.BlockSpec((h, ro), lambda i: (0, 0)),
        ],
        out_specs=pl.BlockSpec((block_n, ro), lambda i: (i, 0)),
        out_shape=jax.ShapeDtypeStruct((n, ro), jnp.bfloat16),
    )(part, bias.reshape(1, h), wt)


def _make_prep(E, R, N):
    EP = E // NW
    NSUB = EP // SUB
    EPA = (EP // 128 + 1) * 128
    mesh = plsc.VectorSubcoreMesh(core_axis_name="c", subcore_axis_name="s")

    @functools.partial(
        pl.kernel,
        mesh=mesh,
        out_type=(
            jax.ShapeDtypeStruct((NW, NSUB, SUB), jnp.int32),
            jax.ShapeDtypeStruct((NW, NSUB, SUB), jnp.int32),
        ),
        scratch_types=[
            pltpu.VMEM((EPA,), jnp.int32),
            pltpu.VMEM((EPA,), jnp.int32),
            pltpu.VMEM((EP,), jnp.int32),
            pltpu.VMEM((NSUB, SUB), jnp.int32),
            pltpu.VMEM((NSUB, SUB), jnp.int32),
        ],
    )
    def prep(ei_hbm, ety_hbm, gidx_out, dst_out,
             srcv, dstv, etyv, gidx2d, dst2d):
        wid = lax.axis_index("c") * NS + lax.axis_index("s")
        base = wid * EP
        rem = lax.rem(base, 128)
        al = pl.multiple_of(base - rem, 128)
        pltpu.sync_copy(ei_hbm.at[0].at[pl.ds(al, EPA)], srcv)
        pltpu.sync_copy(ei_hbm.at[1].at[pl.ds(al, EPA)], dstv)
        pltpu.sync_copy(ety_hbm.at[pl.ds(base, EP)], etyv)

        @pl.loop(0, NSUB)
        def _(j):
            for k in range(SUB // 16):
                off = j * SUB + k * 16
                s = srcv[pl.ds(off + rem, 16)]
                t = etyv[pl.ds(off, 16)]
                gidx2d[j, pl.ds(k * 16, 16)] = s * R + t
                dst2d[j, pl.ds(k * 16, 16)] = dstv[pl.ds(off + rem, 16)]

        pltpu.sync_copy(gidx2d, gidx_out.at[wid])
        pltpu.sync_copy(dst2d, dst_out.at[wid])

    return prep


def _make_scatter(H, N, E):
    EP = E // NW
    NSUB = EP // SUB
    NT8 = ((N // NS) + 7) // 8 * 8
    NTL = N - NT8 * (NS - 1)
    assert NTL > 0 and NTL % 8 == 0
    mesh = plsc.VectorSubcoreMesh(core_axis_name="c", subcore_axis_name="s")

    @functools.partial(
        pl.kernel,
        mesh=mesh,
        out_type=jax.ShapeDtypeStruct((NC, N, H), jnp.bfloat16),
        scratch_types=[
            pltpu.VMEM((NSUB, SUB), jnp.int32),
            pltpu.VMEM((NSUB, SUB), jnp.int32),
            pltpu.VMEM((16, SUB, H), jnp.bfloat16),
            pltpu.VMEM((160, H), jnp.bfloat16),
            pltpu.VMEM_SHARED((N, H), jnp.bfloat16),
            pltpu.SemaphoreType.DMA,
            pltpu.SemaphoreType.DMA,
        ],
        compiler_params=pltpu.CompilerParams(use_tc_tiling_on_sc=False),
    )
    def scatter(table_hbm, gidx_hbm, dst_hbm, out_hbm,
                gidx2d, dst2d, rows, zbuf, acc, sem, psem):
        cid = lax.axis_index("c")
        sid = lax.axis_index("s")
        wid = cid * NS + sid

        pltpu.sync_copy(gidx_hbm.at[wid], gidx2d)
        pltpu.sync_copy(dst_hbm.at[wid], dst2d)

        ZB = zbuf.shape[0]

        @pl.loop(0, ZB)
        def _(i):
            for k in range(H // 32):
                zbuf[i, pl.ds(k * 32, 32)] = jnp.zeros((32,), jnp.bfloat16)

        def zero_span(base_row, count):
            off = 0
            while off < count:
                c = min(ZB, count - off)
                assert c % 8 == 0
                pltpu.sync_copy(zbuf.at[pl.ds(0, c)],
                                acc.at[pl.ds(base_row + off, c)])
                off += c

        @pl.when(sid < NS - 1)
        def _():
            zero_span(sid * NT8, NT8)

        @pl.when(sid == NS - 1)
        def _():
            zero_span((NS - 1) * NT8, NTL)
        plsc.subcore_barrier()

        def fire_g(j, b):
            pltpu.async_copy(table_hbm.at[gidx2d.at[j]], rows.at[b], sem)

        def drain_g(j, b):
            pltpu.make_async_copy(table_hbm.at[gidx2d.at[j]], rows.at[b],
                                  sem).wait()

        def fire_p(j, b):
            pltpu.async_copy(rows.at[b], acc.at[dst2d.at[j]], psem, add=True)

        def drain_p(j, b):
            pltpu.make_async_copy(rows.at[b], acc.at[dst2d.at[j]],
                                  psem).wait()

        NB = rows.shape[0]
        TAIL = NSUB % NB
        assert NSUB >= 2 * NB
        for b in range(NB):
            fire_g(b, b)

        @pl.loop(0, NSUB // NB - 1)
        def _(jj):
            j0 = jj * NB
            for b in range(NB):
                drain_g(j0 + b, b)
                fire_p(j0 + b, b)
            for b in range(NB):
                drain_p(j0 + b, b)
                fire_g(j0 + NB + b, b)

        j0 = NSUB - TAIL - NB
        for b in range(NB):
            drain_g(j0 + b, b)
            fire_p(j0 + b, b)
        for b in range(NB):
            drain_p(j0 + b, b)
            if b < TAIL:
                fire_g(j0 + NB + b, b)
        for b in range(TAIL):
            drain_g(j0 + NB + b, b)
            fire_p(j0 + NB + b, b)
        for b in range(TAIL):
            drain_p(j0 + NB + b, b)
        plsc.subcore_barrier()

        @pl.when(sid < NS - 1)
        def _():
            pltpu.sync_copy(acc.at[pl.ds(sid * NT8, NT8)],
                            out_hbm.at[cid].at[pl.ds(sid * NT8, NT8)])

        @pl.when(sid == NS - 1)
        def _():
            pltpu.sync_copy(acc.at[pl.ds((NS - 1) * NT8, NTL)],
                            out_hbm.at[cid].at[pl.ds((NS - 1) * NT8, NTL)])

    return scatter


def _head_body(p_ref, bias_ref, gw_ref, gb_ref, f1w_ref, f1b_ref,
               f2w_ref, f2b_ref, f3w_ref, f3b_ref, out_ref):
    h = (p_ref[0].astype(jnp.float32) + p_ref[1].astype(jnp.float32)
         + bias_ref[...])
    g = jnp.sum(h * gw_ref[...], axis=1, keepdims=True) + gb_ref[0, 0]
    m = jnp.max(g)
    e = jnp.exp(g - m)
    w = e / jnp.sum(e)
    ro = jnp.sum(h * w, axis=0, keepdims=True)
    z = jnp.maximum(jnp.dot(ro, f1w_ref[...],
                            preferred_element_type=jnp.float32)
                    + f1b_ref[...], 0.0)
    z = jnp.maximum(jnp.dot(z, f2w_ref[...],
                            preferred_element_type=jnp.float32)
                    + f2b_ref[...], 0.0)
    z = jnp.dot(z, f3w_ref[...], preferred_element_type=jnp.float32) \
        + f3b_ref[...]
    out_ref[...] = 1.0 / (1.0 + jnp.exp(-z))


def _head(part, bias, gate_w, gate_b, fc1_w, fc1_b, fc2_w, fc2_b, fc3_w, fc3_b):
    _, n, h = part.shape
    return pl.pallas_call(
        _head_body,
        out_shape=jax.ShapeDtypeStruct((1, 1), jnp.float32),
    )(part, bias.reshape(1, h), gate_w.reshape(1, h), gate_b.reshape(1, 1),
      fc1_w, fc1_b.reshape(1, -1), fc2_w, fc2_b.reshape(1, -1),
      fc3_w, fc3_b.reshape(1, 1))


def kernel(x, edge_index, etype, bases0, comp0, bias0, bases1, comp1, bias1,
           bases2, comp2, bias2, gate_w, gate_b, fc1_w, fc1_b, fc2_w, fc2_b,
           fc3_w, fc3_b):
    n, d = x.shape
    e = etype.shape[0]
    r = comp0.shape[0]
    h = bases0.shape[2]
    o = bases2.shape[2]

    wt0, wt1, wt2 = _make_weights(comp0, bases0, comp1, bases1, comp2, bases2)
    gidx2d, dst2d = _make_prep(e, r, n)(edge_index.astype(jnp.int32),
                                        etype.astype(jnp.int32))
    scatter = _make_scatter(h, n, e)

    t0 = _mm0(x, wt0, block_n=2000).reshape(n * r, h)
    p0 = scatter(t0, gidx2d, dst2d)
    t1 = _mm12(p0, bias0, wt1, block_n=2000).reshape(n * r, h)
    p1 = scatter(t1, gidx2d, dst2d)
    t2 = _mm12(p1, bias1, wt2, block_n=2000).reshape(n * r, o)
    p2 = scatter(t2, gidx2d, dst2d)

    out = _head(p2, bias2, gate_w, gate_b, fc1_w, fc1_b, fc2_w, fc2_b,
                fc3_w, fc3_b)
    return out.reshape((1,))
